# baseline (speedup 1.0000x reference)
import functools

import jax
import jax.numpy as jnp
from jax import lax
from jax.experimental import pallas as pl
from jax.experimental.pallas import tpu as pltpu
from jax.experimental.pallas import tpu_sc as plsc

N_NODES = 10000
N_PAD = 10240
N_EDGES = 160000
FC = 128
NC = 2
NS = 16
ROWS_PER_TILE = N_PAD // NS

_sc_mesh_cache = []


def _sc_mesh():
    if not _sc_mesh_cache:
        _sc_mesh_cache.append(
            plsc.VectorSubcoreMesh(core_axis_name="c", subcore_axis_name="s")
        )
    return _sc_mesh_cache[0]


def _sc_counts(dst, zeros128, ones128):
    block_e = 200
    e_per_core = N_EDGES // NC
    e_per_tile = e_per_core // NS
    n_blocks = e_per_tile // block_e

    @functools.partial(
        pl.kernel,
        out_type=jax.ShapeDtypeStruct((NC * N_PAD, FC), jnp.float32),
        mesh=_sc_mesh(),
        scratch_types=[
            pltpu.VMEM_SHARED((N_PAD, FC), jnp.float32),
            pltpu.VMEM((block_e, FC), jnp.float32),
            pltpu.VMEM((block_e,), jnp.int32),
        ],
    )
    def k(dst_hbm, zeros_hbm, ones_hbm, out_hbm, acc, ones_v, idx_d):
        core = lax.axis_index("c")
        sid = lax.axis_index("s")
        row0 = sid * ROWS_PER_TILE

        pltpu.sync_copy(ones_hbm, ones_v)
        pltpu.sync_copy(
            zeros_hbm.at[pl.ds(row0, ROWS_PER_TILE)],
            acc.at[pl.ds(row0, ROWS_PER_TILE)],
        )
        plsc.subcore_barrier()

        @pl.loop(0, n_blocks)
        def _(b):
            e0 = core * e_per_core + sid * e_per_tile + b * block_e
            pltpu.sync_copy(dst_hbm.at[pl.ds(e0, block_e)], idx_d)
            pltpu.sync_copy(ones_v, acc.at[idx_d], add=True)

        plsc.subcore_barrier()
        pltpu.sync_copy(
            acc.at[pl.ds(row0, ROWS_PER_TILE)],
            out_hbm.at[pl.ds(core * N_PAD + row0, ROWS_PER_TILE)],
        )

    return k(dst, zeros128, ones128).reshape(NC, N_PAD, FC)


def _sc_segsum(chunks, ei4, eit, zeros128, block_e=80, nbuf=4):
    n_chunks = len(chunks)
    cpc = n_chunks // NC
    e_per_tile = N_EDGES // NS
    n_blocks = e_per_tile // block_e
    tail_e = e_per_tile - n_blocks * block_e
    n_rounds = n_blocks // nbuf
    rem = n_blocks - n_rounds * nbuf

    @functools.partial(
        pl.kernel,
        out_type=[
            jax.ShapeDtypeStruct((N_PAD, FC), jnp.float32)
            for _ in range(n_chunks)
        ],
        mesh=_sc_mesh(),
        scratch_types=(
            [pltpu.VMEM_SHARED((N_PAD, FC), jnp.float32)]
            + [pltpu.VMEM((2, block_e), jnp.int32) for _ in range(nbuf)]
            + [pltpu.VMEM((block_e, FC), jnp.float32) for _ in range(nbuf)]
            + [pltpu.VMEM((2, max(tail_e, 8)), jnp.int32)]
            + [pltpu.SemaphoreType.DMA for _ in range(2 * nbuf)]
        ),
    )
    def k(*refs):
        h_refs = refs[:n_chunks]
        ei_hbm, eit_hbm, zeros_hbm = refs[n_chunks:n_chunks + 3]
        out_refs = refs[n_chunks + 3:2 * n_chunks + 3]
        rest = refs[2 * n_chunks + 3:]
        acc = rest[0]
        ibuf = rest[1:1 + nbuf]
        rows = rest[1 + nbuf:1 + 2 * nbuf]
        ibt = rest[1 + 2 * nbuf]
        gsems = rest[2 + 2 * nbuf:2 + 3 * nbuf]
        isems = rest[2 + 3 * nbuf:2 + 4 * nbuf]
        core = lax.axis_index("c")
        sid = lax.axis_index("s")
        row0 = sid * ROWS_PER_TILE

        def process_chunk(ci):
            h = h_refs[ci]
            pltpu.sync_copy(
                zeros_hbm.at[pl.ds(row0, ROWS_PER_TILE)],
                acc.at[pl.ds(row0, ROWS_PER_TILE)],
            )
            plsc.subcore_barrier()

            def start_idx(b, k):
                pltpu.async_copy(ei_hbm.at[sid, b], ibuf[k], isems[k])

            def wait_idx(k):
                pltpu.make_async_copy(ei_hbm.at[0, 0], ibuf[k],
                                      isems[k]).wait()

            def start_gather(b, k):
                pltpu.async_copy(h.at[ibuf[k].at[0]], rows[k], gsems[k])

            def wait_gather(k):
                pltpu.make_async_copy(h.at[pl.ds(0, block_e)],
                                      rows[k], gsems[k]).wait()

            def scatter(b, k):
                pltpu.sync_copy(rows[k], acc.at[ibuf[k].at[1]], add=True)

            for k in range(nbuf):
                start_idx(k, k)
                wait_idx(k)
                start_gather(k, k)

            @pl.loop(0, n_rounds)
            def _(r):
                b0 = r * nbuf
                for k in range(nbuf):
                    b = b0 + k
                    wait_gather(k)
                    scatter(b, k)

                    @pl.when(b + nbuf < n_blocks)
                    def _():
                        start_idx(b + nbuf, k)
                        wait_idx(k)
                        start_gather(b + nbuf, k)

            for k in range(rem):
                b = n_rounds * nbuf + k
                wait_gather(k)
                scatter(b, k)

            if tail_e:
                pltpu.sync_copy(eit_hbm.at[sid], ibt)
                pltpu.sync_copy(h.at[ibt.at[0]], rows[0].at[pl.ds(0, tail_e)])
                pltpu.sync_copy(rows[0].at[pl.ds(0, tail_e)],
                                acc.at[ibt.at[1]], add=True)

            plsc.subcore_barrier()
            pltpu.sync_copy(
                acc.at[pl.ds(row0, ROWS_PER_TILE)],
                out_refs[ci].at[pl.ds(row0, ROWS_PER_TILE)],
            )
            plsc.subcore_barrier()

        for cc in range(cpc):
            @pl.when(core == 0)
            def _():
                process_chunk(cc)

            @pl.when(core == 1)
            def _():
                process_chunk(cpc + cc)

    return k(*chunks, ei4, eit, zeros128)


def _tc_self(chunks, wrT, b, bm=1000):
    n_chunks = len(chunks)
    d_out = wrT.shape[1]
    n_blocks = N_NODES // bm

    def body(*refs):
        x_refs = refs[:n_chunks]
        wr_ref, b_ref, o_ref = refs[n_chunks:]
        wr = wr_ref[...]
        acc = b_ref[...].astype(jnp.float32) + jnp.zeros((bm, d_out), jnp.float32)
        for c in range(n_chunks):
            acc += jnp.dot(x_refs[c][...], wr[c * FC:(c + 1) * FC, :],
                           preferred_element_type=jnp.float32)
        o_ref[...] = acc

    in_specs = (
        [pl.BlockSpec((bm, FC), lambda i: (i, 0)) for _ in range(n_chunks)]
        + [
            pl.BlockSpec((n_chunks * FC, d_out), lambda i: (0, 0)),
            pl.BlockSpec((1, d_out), lambda i: (0, 0)),
        ]
    )
    return pl.pallas_call(
        body,
        grid=(n_blocks,),
        in_specs=in_specs,
        out_specs=pl.BlockSpec((bm, d_out), lambda i: (i, 0)),
        out_shape=jax.ShapeDtypeStruct((N_NODES, d_out), jnp.float32),
    )(*chunks, wrT, b.reshape(1, d_out))


def _tc_combine(aggs, cnt2, wlT, self_term, last, bm=1000):
    n_chunks = len(aggs)
    d_out = wlT.shape[1]
    n_blocks = N_NODES // bm
    out_chunks = d_out // FC

    def body(*refs):
        a_refs = refs[:n_chunks]
        ca_ref, cb_ref, wl_ref, s_ref = refs[n_chunks:n_chunks + 4]
        o_refs = refs[n_chunks + 4:]
        cnt = ca_ref[0, :, :1] + cb_ref[0, :, :1]
        inv = 1.0 / jnp.maximum(cnt, 1.0)
        wl = wl_ref[...]
        acc = s_ref[...]
        for c in range(n_chunks):
            acc += jnp.dot(
                a_refs[c][...] * inv,
                wl[c * FC:(c + 1) * FC, :],
                preferred_element_type=jnp.float32,
            )
        if last:
            h = jnp.maximum(acc, 0.0)
            m = jnp.max(h, axis=1, keepdims=True)
            lse = jnp.log(jnp.sum(jnp.exp(h - m), axis=1, keepdims=True)) + m
            o_refs[0][...] = h - lse
        else:
            for c in range(out_chunks):
                o_refs[c][...] = acc[:, c * FC:(c + 1) * FC]

    in_specs = (
        [pl.BlockSpec((bm, FC), lambda i: (i, 0)) for _ in range(n_chunks)]
        + [
            pl.BlockSpec((1, bm, FC), lambda i: (0, i, 0)),
            pl.BlockSpec((1, bm, FC), lambda i: (1, i, 0)),
            pl.BlockSpec((n_chunks * FC, d_out), lambda i: (0, 0)),
            pl.BlockSpec((bm, d_out), lambda i: (i, 0)),
        ]
    )
    if last:
        out_specs = pl.BlockSpec((bm, d_out), lambda i: (i, 0))
        out_shape = jax.ShapeDtypeStruct((N_NODES, d_out), jnp.float32)
    else:
        out_specs = [pl.BlockSpec((bm, FC), lambda i: (i, 0))
                     for _ in range(out_chunks)]
        out_shape = [jax.ShapeDtypeStruct((N_NODES, FC), jnp.float32)
                     for _ in range(out_chunks)]
    out = pl.pallas_call(
        body,
        grid=(n_blocks,),
        in_specs=in_specs,
        out_specs=out_specs,
        out_shape=out_shape,
    )(*aggs, cnt2, cnt2, wlT, self_term)
    return out


def kernel(x, edge_index, W_l0, b_l0, W_r0, W_l1, b_l1, W_r1, W_l2, b_l2, W_r2):
    src = edge_index[0].astype(jnp.int32)
    dst = edge_index[1].astype(jnp.int32)
    block_e = 80
    ept = N_EDGES // NS
    nb = ept // block_e
    tail_e = ept - nb * block_e
    ei = jnp.stack([src, dst]).reshape(2, NS, ept)
    ei4 = ei[:, :, :nb * block_e].reshape(2, NS, nb, block_e).transpose(1, 2, 0, 3)
    if tail_e:
        eit = ei[:, :, nb * block_e:].transpose(1, 0, 2)
    else:
        eit = jnp.zeros((NS, 2, 8), jnp.int32)
    zeros128 = jnp.zeros((N_PAD, FC), jnp.float32)

    cnt2 = _sc_counts(dst, zeros128, jnp.ones((200, FC), jnp.float32))

    chunks = [x[:, c * FC:(c + 1) * FC] for c in range(x.shape[1] // FC)]
    layers = [(W_l0, b_l0, W_r0), (W_l1, b_l1, W_r1), (W_l2, b_l2, W_r2)]
    for li, (wl, b, wr) in enumerate(layers):
        self_term = _tc_self(chunks, wr.T, b)
        aggs = _sc_segsum(chunks, ei4, eit, zeros128, block_e=block_e)
        chunks = _tc_combine(aggs, cnt2, wl.T, self_term, last=(li == 2))
    return chunks


# speedup vs baseline: 1.1636x; 1.1636x over previous
import functools

import jax
import jax.numpy as jnp
from jax import lax
from jax.experimental import pallas as pl
from jax.experimental.pallas import tpu as pltpu
from jax.experimental.pallas import tpu_sc as plsc

N_NODES = 10000
N_PAD = 10240
N_EDGES = 160000
FC = 128
NC = 2
NS = 16
ROWS_PER_TILE = N_PAD // NS

_sc_mesh_cache = []


def _sc_mesh():
    if not _sc_mesh_cache:
        _sc_mesh_cache.append(
            plsc.VectorSubcoreMesh(core_axis_name="c", subcore_axis_name="s")
        )
    return _sc_mesh_cache[0]


def _sc_counts(dst, zeros128, ones128):
    block_e = 200
    e_per_core = N_EDGES // NC
    e_per_tile = e_per_core // NS
    n_blocks = e_per_tile // block_e

    @functools.partial(
        pl.kernel,
        out_type=jax.ShapeDtypeStruct((NC * N_PAD, FC), jnp.float32),
        mesh=_sc_mesh(),
        scratch_types=[
            pltpu.VMEM_SHARED((N_PAD, FC), jnp.float32),
            pltpu.VMEM((block_e, FC), jnp.float32),
            pltpu.VMEM((block_e,), jnp.int32),
        ],
    )
    def k(dst_hbm, zeros_hbm, ones_hbm, out_hbm, acc, ones_v, idx_d):
        core = lax.axis_index("c")
        sid = lax.axis_index("s")
        row0 = sid * ROWS_PER_TILE

        pltpu.sync_copy(ones_hbm, ones_v)
        pltpu.sync_copy(
            zeros_hbm.at[pl.ds(row0, ROWS_PER_TILE)],
            acc.at[pl.ds(row0, ROWS_PER_TILE)],
        )
        plsc.subcore_barrier()

        @pl.loop(0, n_blocks)
        def _(b):
            e0 = core * e_per_core + sid * e_per_tile + b * block_e
            pltpu.sync_copy(dst_hbm.at[pl.ds(e0, block_e)], idx_d)
            pltpu.sync_copy(ones_v, acc.at[idx_d], add=True)

        plsc.subcore_barrier()
        pltpu.sync_copy(
            acc.at[pl.ds(row0, ROWS_PER_TILE)],
            out_hbm.at[pl.ds(core * N_PAD + row0, ROWS_PER_TILE)],
        )

    return k(dst, zeros128, ones128).reshape(NC, N_PAD, FC)


def _sc_segsum(chunks, src, dst, zeros128, block_e=184):
    n_chunks = len(chunks)
    cpc = n_chunks // NC
    e_per_tile = N_EDGES // NS
    n_blocks = e_per_tile // block_e
    tail_e = e_per_tile - n_blocks * block_e
    n_pairs = n_blocks // 2
    odd = n_blocks % 2

    @functools.partial(
        pl.kernel,
        out_type=[
            jax.ShapeDtypeStruct((N_PAD, FC), jnp.float32)
            for _ in range(n_chunks)
        ],
        mesh=_sc_mesh(),
        scratch_types=[
            pltpu.VMEM_SHARED((N_PAD, FC), jnp.float32),
            pltpu.VMEM((block_e,), jnp.int32),
            pltpu.VMEM((block_e,), jnp.int32),
            pltpu.VMEM((block_e,), jnp.int32),
            pltpu.VMEM((block_e,), jnp.int32),
            pltpu.VMEM((block_e, FC), jnp.float32),
            pltpu.VMEM((block_e, FC), jnp.float32),
            pltpu.VMEM((max(tail_e, 8),), jnp.int32),
            pltpu.VMEM((max(tail_e, 8),), jnp.int32),
            pltpu.SemaphoreType.DMA,
            pltpu.SemaphoreType.DMA,
            pltpu.SemaphoreType.DMA,
            pltpu.SemaphoreType.DMA,
        ],
    )
    def k(*refs):
        h_refs = refs[:n_chunks]
        src_hbm, dst_hbm, zeros_hbm = refs[n_chunks:n_chunks + 3]
        out_refs = refs[n_chunks + 3:2 * n_chunks + 3]
        (acc, sb0, sb1, db0, db1, rows0, rows1, sbt, dbt,
         gsem0, gsem1, isem0, isem1) = refs[2 * n_chunks + 3:]
        core = lax.axis_index("c")
        sid = lax.axis_index("s")
        row0 = sid * ROWS_PER_TILE
        e_base = sid * e_per_tile
        sbuf = (sb0, sb1)
        dbuf = (db0, db1)
        rows = (rows0, rows1)
        gsems = (gsem0, gsem1)
        isems = (isem0, isem1)

        def process_chunk(ci):
            h = h_refs[ci]
            pltpu.sync_copy(
                zeros_hbm.at[pl.ds(row0, ROWS_PER_TILE)],
                acc.at[pl.ds(row0, ROWS_PER_TILE)],
            )
            plsc.subcore_barrier()

            def start_idx(b, k):
                e0 = e_base + b * block_e
                pltpu.async_copy(src_hbm.at[pl.ds(e0, block_e)],
                                 sbuf[k], isems[k])
                pltpu.async_copy(dst_hbm.at[pl.ds(e0, block_e)],
                                 dbuf[k], isems[k])

            def wait_idx(k):
                pltpu.make_async_copy(src_hbm.at[pl.ds(0, block_e)],
                                      sbuf[k], isems[k]).wait()
                pltpu.make_async_copy(dst_hbm.at[pl.ds(0, block_e)],
                                      dbuf[k], isems[k]).wait()

            def start_gather(k):
                pltpu.async_copy(h.at[sbuf[k]], rows[k], gsems[k])

            def wait_gather(k):
                pltpu.make_async_copy(h.at[pl.ds(0, block_e)],
                                      rows[k], gsems[k]).wait()

            def scatter(k):
                pltpu.sync_copy(rows[k], acc.at[dbuf[k]], add=True)

            start_idx(0, 0)
            wait_idx(0)
            start_gather(0)
            if n_blocks > 1:
                start_idx(1, 1)
                wait_idx(1)
                start_gather(1)

            def stage(b, k):
                wait_gather(k)
                scatter(k)

                @pl.when(b + 2 < n_blocks)
                def _():
                    start_idx(b + 2, k)
                    wait_idx(k)
                    start_gather(k)

            @pl.loop(0, n_pairs)
            def _(p):
                stage(2 * p, 0)
                stage(2 * p + 1, 1)

            if odd:
                wait_gather(0)
                scatter(0)

            if tail_e:
                e0 = e_base + n_blocks * block_e
                pltpu.sync_copy(src_hbm.at[pl.ds(e0, tail_e)], sbt)
                pltpu.sync_copy(dst_hbm.at[pl.ds(e0, tail_e)], dbt)
                pltpu.sync_copy(h.at[sbt], rows0.at[pl.ds(0, tail_e)])
                pltpu.sync_copy(rows0.at[pl.ds(0, tail_e)],
                                acc.at[dbt], add=True)

            plsc.subcore_barrier()
            pltpu.sync_copy(
                acc.at[pl.ds(row0, ROWS_PER_TILE)],
                out_refs[ci].at[pl.ds(row0, ROWS_PER_TILE)],
            )
            plsc.subcore_barrier()

        for cc in range(cpc):
            @pl.when(core == 0)
            def _():
                process_chunk(cc)

            @pl.when(core == 1)
            def _():
                process_chunk(cpc + cc)

    return k(*chunks, src, dst, zeros128)


def _tc_self(chunks, wrT, b, bm=1000):
    n_chunks = len(chunks)
    d_out = wrT.shape[1]
    n_blocks = N_NODES // bm

    def body(*refs):
        x_refs = refs[:n_chunks]
        wr_ref, b_ref, o_ref = refs[n_chunks:]
        wr = wr_ref[...]
        acc = b_ref[...].astype(jnp.float32) + jnp.zeros((bm, d_out), jnp.float32)
        for c in range(n_chunks):
            acc += jnp.dot(x_refs[c][...], wr[c * FC:(c + 1) * FC, :],
                           preferred_element_type=jnp.float32)
        o_ref[...] = acc

    in_specs = (
        [pl.BlockSpec((bm, FC), lambda i: (i, 0)) for _ in range(n_chunks)]
        + [
            pl.BlockSpec((n_chunks * FC, d_out), lambda i: (0, 0)),
            pl.BlockSpec((1, d_out), lambda i: (0, 0)),
        ]
    )
    return pl.pallas_call(
        body,
        grid=(n_blocks,),
        in_specs=in_specs,
        out_specs=pl.BlockSpec((bm, d_out), lambda i: (i, 0)),
        out_shape=jax.ShapeDtypeStruct((N_NODES, d_out), jnp.float32),
    )(*chunks, wrT, b.reshape(1, d_out))


def _tc_combine(aggs, cnt2, wlT, self_term, last, bm=1000):
    n_chunks = len(aggs)
    d_out = wlT.shape[1]
    n_blocks = N_NODES // bm
    out_chunks = d_out // FC

    def body(*refs):
        a_refs = refs[:n_chunks]
        ca_ref, cb_ref, wl_ref, s_ref = refs[n_chunks:n_chunks + 4]
        o_refs = refs[n_chunks + 4:]
        cnt = ca_ref[0, :, :1] + cb_ref[0, :, :1]
        inv = 1.0 / jnp.maximum(cnt, 1.0)
        wl = wl_ref[...]
        acc = s_ref[...]
        for c in range(n_chunks):
            acc += jnp.dot(
                a_refs[c][...] * inv,
                wl[c * FC:(c + 1) * FC, :],
                preferred_element_type=jnp.float32,
            )
        if last:
            h = jnp.maximum(acc, 0.0)
            m = jnp.max(h, axis=1, keepdims=True)
            lse = jnp.log(jnp.sum(jnp.exp(h - m), axis=1, keepdims=True)) + m
            o_refs[0][...] = h - lse
        else:
            for c in range(out_chunks):
                o_refs[c][...] = acc[:, c * FC:(c + 1) * FC]

    in_specs = (
        [pl.BlockSpec((bm, FC), lambda i: (i, 0)) for _ in range(n_chunks)]
        + [
            pl.BlockSpec((1, bm, FC), lambda i: (0, i, 0)),
            pl.BlockSpec((1, bm, FC), lambda i: (1, i, 0)),
            pl.BlockSpec((n_chunks * FC, d_out), lambda i: (0, 0)),
            pl.BlockSpec((bm, d_out), lambda i: (i, 0)),
        ]
    )
    if last:
        out_specs = pl.BlockSpec((bm, d_out), lambda i: (i, 0))
        out_shape = jax.ShapeDtypeStruct((N_NODES, d_out), jnp.float32)
    else:
        out_specs = [pl.BlockSpec((bm, FC), lambda i: (i, 0))
                     for _ in range(out_chunks)]
        out_shape = [jax.ShapeDtypeStruct((N_NODES, FC), jnp.float32)
                     for _ in range(out_chunks)]
    out = pl.pallas_call(
        body,
        grid=(n_blocks,),
        in_specs=in_specs,
        out_specs=out_specs,
        out_shape=out_shape,
    )(*aggs, cnt2, cnt2, wlT, self_term)
    return out


def kernel(x, edge_index, W_l0, b_l0, W_r0, W_l1, b_l1, W_r1, W_l2, b_l2, W_r2):
    src = edge_index[0].astype(jnp.int32)
    dst = edge_index[1].astype(jnp.int32)
    zeros128 = jnp.zeros((N_PAD, FC), jnp.float32)

    cnt2 = _sc_counts(dst, zeros128, jnp.ones((200, FC), jnp.float32))

    chunks = [x[:, c * FC:(c + 1) * FC] for c in range(x.shape[1] // FC)]
    layers = [(W_l0, b_l0, W_r0), (W_l1, b_l1, W_r1), (W_l2, b_l2, W_r2)]
    for li, (wl, b, wr) in enumerate(layers):
        self_term = _tc_self(chunks, wr.T, b)
        aggs = _sc_segsum(chunks, src, dst, zeros128)
        chunks = _tc_combine(aggs, cnt2, wl.T, self_term, last=(li == 2))
    return chunks
